# baseline (speedup 1.0000x reference)
import functools

import jax
import jax.numpy as jnp
from jax import lax
from jax.experimental import pallas as pl
from jax.experimental.pallas import tpu as pltpu
from jax.experimental.pallas import tpu_sc as plsc

NUM_CORES = 2
NUM_SUBCORES = 16
NUM_WORKERS = NUM_CORES * NUM_SUBCORES
CHUNK = 80


def _make_segsum_sc(N, D, deg):
    n_chunks = N // CHUNK
    assert n_chunks * CHUNK == N
    ch_base = n_chunks // NUM_WORKERS
    n_hi = n_chunks - ch_base * NUM_WORKERS
    ch_hi = ch_base + 1
    max_nodes = ch_hi * CHUNK

    mesh = plsc.VectorSubcoreMesh(
        core_axis_name="c", subcore_axis_name="s",
        num_cores=NUM_CORES, num_subcores=NUM_SUBCORES)

    N_ADD = 2
    red_base = N_ADD * CHUNK
    NBUF = 4

    @functools.partial(
        pl.kernel,
        out_type=jax.ShapeDtypeStruct((N, D), jnp.float32),
        mesh=mesh,
        scratch_types=[
            pltpu.VMEM((N_ADD, deg, CHUNK), jnp.int32),
            pltpu.VMEM(((ch_hi - N_ADD) * CHUNK * deg,), jnp.int32),
            pltpu.VMEM((max_nodes, D), jnp.float32),
        ] + [pltpu.VMEM((deg, D), jnp.float32)] * NBUF
          + [pltpu.SemaphoreType.DMA] * (N_ADD + NBUF),
    )
    def segsum(x_hbm, nt_hbm, nids_hbm, s_hbm, idxT, idxraw, out_all, *rest):
        gbufs = rest[:NBUF]
        asems = rest[NBUF:NBUF + N_ADD]
        gsems = rest[NBUF + N_ADD:]
        c = lax.axis_index("c")
        s = lax.axis_index("s")
        w = c * NUM_SUBCORES + s
        is_hi = w < n_hi
        chunk0 = w * ch_base + jnp.minimum(w, n_hi)
        chunks_w = jnp.where(is_hi, ch_hi, ch_base)
        base = chunk0 * CHUNK
        n_red = (chunks_w - N_ADD) * CHUNK

        pltpu.sync_copy(nt_hbm.at[pl.ds(chunk0, N_ADD)], idxT)
        raw0 = (base + red_base) * deg

        @pl.when(is_hi)
        def _():
            pltpu.sync_copy(
                nids_hbm.at[pl.ds(raw0, (ch_hi - N_ADD) * CHUNK * deg)],
                idxraw)

        @pl.when(jnp.logical_not(is_hi))
        def _():
            pltpu.sync_copy(
                nids_hbm.at[pl.ds(raw0, (ch_base - N_ADD) * CHUNK * deg)],
                idxraw.at[pl.ds(0, (ch_base - N_ADD) * CHUNK * deg)])

        zeros_f = jnp.zeros((16,), jnp.float32)

        def zrow(i, carry):
            for cc in range(D // 16):
                out_all[i, pl.ds(cc * 16, 16)] = zeros_f
            return carry

        lax.fori_loop(0, red_base, zrow, 0)

        for ch in range(N_ADD):
            dst = out_all.at[pl.ds(ch * CHUNK, CHUNK)]
            for g in range(deg):
                pltpu.async_copy(x_hbm.at[idxT.at[ch, g]], dst, asems[ch],
                                 add=True)

        def start_gather(j, b):
            pltpu.async_copy(x_hbm.at[idxraw.at[pl.ds(j * deg, deg)]],
                             gbufs[b], gsems[b])

        def wait_gather(b):
            pltpu.make_async_copy(x_hbm.at[idxraw.at[pl.ds(0, deg)]],
                                  gbufs[b], gsems[b]).wait()

        def reduce_into(b, j):
            buf = gbufs[b]
            for cc in range(D // 16):
                sl = pl.ds(cc * 16, 16)
                accs = [buf[k, sl] for k in range(4)]
                for r in range(4, deg):
                    accs[r % 4] = accs[r % 4] + buf[r, sl]
                out_all[red_base + j, sl] = (accs[0] + accs[1]) + (
                    accs[2] + accs[3])

        for k in range(NBUF - 1):
            start_gather(k, k)

        def body(io, carry):
            i = NBUF * io
            for k in range(NBUF):
                nxt = i + k + NBUF - 1

                @pl.when(nxt < n_red)
                def _(nxt=nxt, k=k):
                    start_gather(nxt, (k + NBUF - 1) % NBUF)

                wait_gather(k)
                reduce_into(k, i + k)
            return carry

        lax.fori_loop(0, n_red // NBUF, body, 0)

        for ch in range(N_ADD):
            dst = out_all.at[pl.ds(ch * CHUNK, CHUNK)]
            for g in range(deg):
                pltpu.make_async_copy(x_hbm.at[idxT.at[ch, g]], dst,
                                      asems[ch]).wait()

        @pl.when(is_hi)
        def _():
            pltpu.sync_copy(out_all, s_hbm.at[pl.ds(base, max_nodes)])

        @pl.when(jnp.logical_not(is_hi))
        def _():
            pltpu.sync_copy(out_all.at[pl.ds(0, ch_base * CHUNK)],
                            s_hbm.at[pl.ds(base, ch_base * CHUNK)])

    return segsum


def _tc_fuse_body(x_ref, s_ref, wg_ref, wc_ref, b_ref, o_ref):
    dn = (((1,), (1,)), ((), ()))
    o = lax.dot_general(x_ref[...], wg_ref[...], dn,
                        preferred_element_type=jnp.float32)
    o = o + lax.dot_general(s_ref[...], wc_ref[...], dn,
                            preferred_element_type=jnp.float32)
    o = o + b_ref[...]
    o_ref[...] = jnp.where(o > 0, o, jnp.exp(jnp.minimum(o, 0.0)) - 1.0)


def _tc_fuse(x, S, Wg, Wc, bias2d):
    N, D = x.shape
    DO = Wg.shape[0]
    blk = 1000
    return pl.pallas_call(
        _tc_fuse_body,
        grid=(N // blk,),
        in_specs=[
            pl.BlockSpec((blk, D), lambda i: (i, 0)),
            pl.BlockSpec((blk, D), lambda i: (i, 0)),
            pl.BlockSpec((DO, D), lambda i: (0, 0)),
            pl.BlockSpec((DO, D), lambda i: (0, 0)),
            pl.BlockSpec((1, DO), lambda i: (0, 0)),
        ],
        out_specs=pl.BlockSpec((blk, DO), lambda i: (i, 0)),
        out_shape=jax.ShapeDtypeStruct((N, DO), jnp.float32),
    )(x, S, Wg, Wc, bias2d)


def kernel(x, W_global, W_local, bias, node_ids, neigh_ids, deg):
    N, D = x.shape
    E = neigh_ids.shape[0]
    deg_static = E // N
    nT = (neigh_ids.astype(jnp.int32)
          .reshape(N // CHUNK, CHUNK, deg_static)
          .transpose(0, 2, 1))
    segsum = _make_segsum_sc(N, D, deg_static)
    S = segsum(x, nT, neigh_ids.astype(jnp.int32))
    inv_deg = 1.0 / jnp.asarray(deg, jnp.float32)
    Wc = (W_local + W_global) * inv_deg
    out = _tc_fuse(x, S, W_global, Wc, bias.reshape(1, -1))
    return out


# speedup vs baseline: 1.0301x; 1.0301x over previous
import functools

import jax
import jax.numpy as jnp
from jax import lax
from jax.experimental import pallas as pl
from jax.experimental.pallas import tpu as pltpu
from jax.experimental.pallas import tpu_sc as plsc

NUM_CORES = 2
NUM_SUBCORES = 16
NUM_WORKERS = NUM_CORES * NUM_SUBCORES
CHUNK = 80


def _make_segsum_sc(N, D, deg):
    n_chunks = N // CHUNK
    assert n_chunks * CHUNK == N
    ch_base = n_chunks // NUM_WORKERS
    n_hi = n_chunks - ch_base * NUM_WORKERS
    ch_hi = ch_base + 1
    max_nodes = ch_hi * CHUNK

    mesh = plsc.VectorSubcoreMesh(
        core_axis_name="c", subcore_axis_name="s",
        num_cores=NUM_CORES, num_subcores=NUM_SUBCORES)

    N_ADD = 2
    red_base = N_ADD * CHUNK
    NBUF = 4

    @functools.partial(
        pl.kernel,
        out_type=jax.ShapeDtypeStruct((N, D), jnp.float32),
        mesh=mesh,
        scratch_types=[
            pltpu.VMEM((N_ADD, deg, CHUNK), jnp.int32),
            pltpu.VMEM(((ch_hi - N_ADD) * CHUNK * deg,), jnp.int32),
            pltpu.VMEM((max_nodes, D), jnp.float32),
        ] + [pltpu.VMEM((deg, D), jnp.float32)] * NBUF
          + [pltpu.SemaphoreType.DMA] * (1 + NBUF),
    )
    def segsum(x_hbm, nt_hbm, nids_hbm, s_hbm, idxT, idxraw, out_all, *rest):
        gbufs = rest[:NBUF]
        asem = rest[NBUF]
        gsems = rest[NBUF + 1:]
        c = lax.axis_index("c")
        s = lax.axis_index("s")
        w = c * NUM_SUBCORES + s
        is_hi = w < n_hi
        chunk0 = w * ch_base + jnp.minimum(w, n_hi)
        chunks_w = jnp.where(is_hi, ch_hi, ch_base)
        base = chunk0 * CHUNK
        n_red = (chunks_w - N_ADD) * CHUNK

        pltpu.sync_copy(nt_hbm.at[pl.ds(chunk0, N_ADD)], idxT)
        raw0 = (base + red_base) * deg

        @pl.when(is_hi)
        def _():
            pltpu.sync_copy(
                nids_hbm.at[pl.ds(raw0, (ch_hi - N_ADD) * CHUNK * deg)],
                idxraw)

        @pl.when(jnp.logical_not(is_hi))
        def _():
            pltpu.sync_copy(
                nids_hbm.at[pl.ds(raw0, (ch_base - N_ADD) * CHUNK * deg)],
                idxraw.at[pl.ds(0, (ch_base - N_ADD) * CHUNK * deg)])

        zeros_f = jnp.zeros((16,), jnp.float32)

        def zrow(i, carry):
            for cc in range(D // 16):
                out_all[i, pl.ds(cc * 16, 16)] = zeros_f
            return carry

        lax.fori_loop(0, red_base, zrow, 0)

        TOTAL_ADD = N_ADD * deg

        def issue_add(sidx):
            ch = sidx // deg
            g = sidx % deg
            dst = out_all.at[pl.ds(ch * CHUNK, CHUNK)]
            pltpu.async_copy(x_hbm.at[idxT.at[ch, g]], dst, asem, add=True)

        def start_gather(j, b):
            pltpu.async_copy(x_hbm.at[idxraw.at[pl.ds(j * deg, deg)]],
                             gbufs[b], gsems[b])

        def wait_gather(b):
            pltpu.make_async_copy(x_hbm.at[idxraw.at[pl.ds(0, deg)]],
                                  gbufs[b], gsems[b]).wait()

        def reduce_into(b, j):
            buf = gbufs[b]
            for cc in range(D // 16):
                sl = pl.ds(cc * 16, 16)
                accs = [buf[k, sl] for k in range(4)]
                for r in range(4, deg):
                    accs[r % 4] = accs[r % 4] + buf[r, sl]
                out_all[red_base + j, sl] = (accs[0] + accs[1]) + (
                    accs[2] + accs[3])

        for k in range(NBUF - 1):
            start_gather(k, k)

        def body(io, carry):
            i = NBUF * io
            for k in range(NBUF):
                nxt = i + k + NBUF - 1

                @pl.when(nxt < n_red)
                def _(nxt=nxt, k=k):
                    start_gather(nxt, (k + NBUF - 1) % NBUF)

                if k % 2 == 0:
                    sidx = (i + k) // 2

                    @pl.when(sidx < TOTAL_ADD)
                    def _(sidx=sidx):
                        issue_add(sidx)

                wait_gather(k)
                reduce_into(k, i + k)
            return carry

        lax.fori_loop(0, n_red // NBUF, body, 0)

        def tail_issue(sidx, carry):
            issue_add(sidx)
            return carry

        lax.fori_loop(jnp.minimum(n_red // 2, TOTAL_ADD), TOTAL_ADD,
                      tail_issue, 0)

        def drain_add(sidx, carry):
            ch = sidx // deg
            g = sidx % deg
            dst = out_all.at[pl.ds(ch * CHUNK, CHUNK)]
            pltpu.make_async_copy(x_hbm.at[idxT.at[ch, g]], dst, asem).wait()
            return carry

        lax.fori_loop(0, TOTAL_ADD, drain_add, 0)

        @pl.when(is_hi)
        def _():
            pltpu.sync_copy(out_all, s_hbm.at[pl.ds(base, max_nodes)])

        @pl.when(jnp.logical_not(is_hi))
        def _():
            pltpu.sync_copy(out_all.at[pl.ds(0, ch_base * CHUNK)],
                            s_hbm.at[pl.ds(base, ch_base * CHUNK)])

    return segsum


def _tc_fuse_body(x_ref, s_ref, wg_ref, wc_ref, b_ref, o_ref):
    dn = (((1,), (1,)), ((), ()))
    o = lax.dot_general(x_ref[...], wg_ref[...], dn,
                        preferred_element_type=jnp.float32)
    o = o + lax.dot_general(s_ref[...], wc_ref[...], dn,
                            preferred_element_type=jnp.float32)
    o = o + b_ref[...]
    o_ref[...] = jnp.where(o > 0, o, jnp.exp(jnp.minimum(o, 0.0)) - 1.0)


def _tc_fuse(x, S, Wg, Wc, bias2d):
    N, D = x.shape
    DO = Wg.shape[0]
    blk = 1000
    return pl.pallas_call(
        _tc_fuse_body,
        grid=(N // blk,),
        in_specs=[
            pl.BlockSpec((blk, D), lambda i: (i, 0)),
            pl.BlockSpec((blk, D), lambda i: (i, 0)),
            pl.BlockSpec((DO, D), lambda i: (0, 0)),
            pl.BlockSpec((DO, D), lambda i: (0, 0)),
            pl.BlockSpec((1, DO), lambda i: (0, 0)),
        ],
        out_specs=pl.BlockSpec((blk, DO), lambda i: (i, 0)),
        out_shape=jax.ShapeDtypeStruct((N, DO), jnp.float32),
    )(x, S, Wg, Wc, bias2d)


def kernel(x, W_global, W_local, bias, node_ids, neigh_ids, deg):
    N, D = x.shape
    E = neigh_ids.shape[0]
    deg_static = E // N
    nT = (neigh_ids.astype(jnp.int32)
          .reshape(N // CHUNK, CHUNK, deg_static)
          .transpose(0, 2, 1))
    segsum = _make_segsum_sc(N, D, deg_static)
    S = segsum(x, nT, neigh_ids.astype(jnp.int32))
    inv_deg = 1.0 / jnp.asarray(deg, jnp.float32)
    Wc = (W_local + W_global) * inv_deg
    out = _tc_fuse(x, S, W_global, Wc, bias.reshape(1, -1))
    return out


# speedup vs baseline: 1.2094x; 1.1741x over previous
import functools

import jax
import jax.numpy as jnp
from jax import lax
from jax.experimental import pallas as pl
from jax.experimental.pallas import tpu as pltpu
from jax.experimental.pallas import tpu_sc as plsc

NUM_CORES = 2
NUM_SUBCORES = 16
NUM_WORKERS = NUM_CORES * NUM_SUBCORES
CHUNK = 80


def _make_segsum_sc(N, D, deg):
    n_chunks = N // CHUNK
    assert n_chunks * CHUNK == N
    ch_base = n_chunks // NUM_WORKERS
    n_hi = n_chunks - ch_base * NUM_WORKERS
    ch_hi = ch_base + 1
    max_nodes = ch_hi * CHUNK

    mesh = plsc.VectorSubcoreMesh(
        core_axis_name="c", subcore_axis_name="s",
        num_cores=NUM_CORES, num_subcores=NUM_SUBCORES)

    @functools.partial(
        pl.kernel,
        out_type=jax.ShapeDtypeStruct((N, D), jnp.float32),
        mesh=mesh,
        scratch_types=[
            pltpu.VMEM((ch_hi, deg, CHUNK), jnp.int32),
            pltpu.VMEM((max_nodes, D), jnp.float32),
        ] + [pltpu.SemaphoreType.DMA] * ch_hi,
    )
    def segsum(x_hbm, nt_hbm, s_hbm, idxT, out_all, *sems):
        c = lax.axis_index("c")
        s = lax.axis_index("s")
        w = c * NUM_SUBCORES + s
        is_hi = w < n_hi
        chunk0 = w * ch_base + jnp.minimum(w, n_hi)
        chunks_w = jnp.where(is_hi, ch_hi, ch_base)
        base = chunk0 * CHUNK

        @pl.when(is_hi)
        def _():
            pltpu.sync_copy(nt_hbm.at[pl.ds(chunk0, ch_hi)], idxT)

        @pl.when(jnp.logical_not(is_hi))
        def _():
            pltpu.sync_copy(nt_hbm.at[pl.ds(chunk0, ch_base)],
                            idxT.at[pl.ds(0, ch_base)])

        zeros_f = jnp.zeros((16,), jnp.float32)

        def zero_chunk(ch):
            def zrow(i, carry):
                for cc in range(D // 16):
                    out_all[i, pl.ds(cc * 16, 16)] = zeros_f
                return carry

            lax.fori_loop(ch * CHUNK, (ch + 1) * CHUNK, zrow, 0)

        def issue(ch, sem):
            dst = out_all.at[pl.ds(ch * CHUNK, CHUNK)]
            for g in range(deg):
                pltpu.async_copy(x_hbm.at[idxT.at[ch, g]], dst, sem, add=True)

        def drain(ch, sem):
            dst = out_all.at[pl.ds(ch * CHUNK, CHUNK)]
            for g in range(deg):
                pltpu.make_async_copy(x_hbm.at[idxT.at[ch, g]], dst,
                                      sem).wait()

        for ch in range(ch_hi):
            @pl.when(ch < chunks_w)
            def _(ch=ch):
                zero_chunk(ch)
                issue(ch, sems[ch])

        for ch in range(ch_hi):
            @pl.when(ch < chunks_w)
            def _(ch=ch):
                drain(ch, sems[ch])

        @pl.when(is_hi)
        def _():
            pltpu.sync_copy(out_all, s_hbm.at[pl.ds(base, max_nodes)])

        @pl.when(jnp.logical_not(is_hi))
        def _():
            pltpu.sync_copy(out_all.at[pl.ds(0, ch_base * CHUNK)],
                            s_hbm.at[pl.ds(base, ch_base * CHUNK)])

    return segsum


def _tc_fuse_body(x_ref, s_ref, wg_ref, wc_ref, b_ref, o_ref):
    dn = (((1,), (1,)), ((), ()))
    o = lax.dot_general(x_ref[...], wg_ref[...], dn,
                        preferred_element_type=jnp.float32)
    o = o + lax.dot_general(s_ref[...], wc_ref[...], dn,
                            preferred_element_type=jnp.float32)
    o = o + b_ref[...]
    o_ref[...] = jnp.where(o > 0, o, jnp.exp(jnp.minimum(o, 0.0)) - 1.0)


def _tc_fuse(x, S, Wg, Wc, bias2d):
    N, D = x.shape
    DO = Wg.shape[0]
    blk = 2000
    return pl.pallas_call(
        _tc_fuse_body,
        grid=(N // blk,),
        in_specs=[
            pl.BlockSpec((blk, D), lambda i: (i, 0)),
            pl.BlockSpec((blk, D), lambda i: (i, 0)),
            pl.BlockSpec((DO, D), lambda i: (0, 0)),
            pl.BlockSpec((DO, D), lambda i: (0, 0)),
            pl.BlockSpec((1, DO), lambda i: (0, 0)),
        ],
        out_specs=pl.BlockSpec((blk, DO), lambda i: (i, 0)),
        out_shape=jax.ShapeDtypeStruct((N, DO), jnp.float32),
    )(x, S, Wg, Wc, bias2d)


def kernel(x, W_global, W_local, bias, node_ids, neigh_ids, deg):
    N, D = x.shape
    E = neigh_ids.shape[0]
    deg_static = E // N
    nT = (neigh_ids.astype(jnp.int32)
          .reshape(N // CHUNK, CHUNK, deg_static)
          .transpose(0, 2, 1))
    segsum = _make_segsum_sc(N, D, deg_static)
    S = segsum(x, nT)
    inv_deg = 1.0 / jnp.asarray(deg, jnp.float32)
    Wc = (W_local + W_global) * inv_deg
    out = _tc_fuse(x, S, W_global, Wc, bias.reshape(1, -1))
    return out


# speedup vs baseline: 1.2170x; 1.0062x over previous
import functools

import jax
import jax.numpy as jnp
from jax import lax
from jax.experimental import pallas as pl
from jax.experimental.pallas import tpu as pltpu
from jax.experimental.pallas import tpu_sc as plsc

NUM_CORES = 2
NUM_SUBCORES = 16
NUM_WORKERS = NUM_CORES * NUM_SUBCORES
CHUNK = 80


def _make_segsum_sc(N, D, deg):
    n_chunks = N // CHUNK
    assert n_chunks * CHUNK == N
    ch_base = n_chunks // NUM_WORKERS
    n_hi = n_chunks - ch_base * NUM_WORKERS
    ch_hi = ch_base + 1
    max_nodes = ch_hi * CHUNK

    mesh = plsc.VectorSubcoreMesh(
        core_axis_name="c", subcore_axis_name="s",
        num_cores=NUM_CORES, num_subcores=NUM_SUBCORES)

    @functools.partial(
        pl.kernel,
        out_type=jax.ShapeDtypeStruct((N, D), jnp.float32),
        mesh=mesh,
        scratch_types=[
            pltpu.VMEM((ch_hi, deg, CHUNK), jnp.int32),
            pltpu.VMEM((max_nodes, D), jnp.float32),
        ] + [pltpu.SemaphoreType.DMA] * (ch_hi + 2),
    )
    def segsum(x_hbm, nt_hbm, s_hbm, idxT, out_all, *sems):
        c = lax.axis_index("c")
        s = lax.axis_index("s")
        w = c * NUM_SUBCORES + s
        is_hi = w < n_hi
        chunk0 = w * ch_base + jnp.minimum(w, n_hi)
        chunks_w = jnp.where(is_hi, ch_hi, ch_base)
        base = chunk0 * CHUNK

        ssem = sems[ch_hi]
        fsem = sems[ch_hi + 1]

        @pl.when(is_hi)
        def _():
            pltpu.async_copy(nt_hbm.at[pl.ds(chunk0, ch_hi)], idxT, ssem)

        @pl.when(jnp.logical_not(is_hi))
        def _():
            pltpu.async_copy(nt_hbm.at[pl.ds(chunk0, ch_base)],
                             idxT.at[pl.ds(0, ch_base)], ssem)

        zeros_f = jnp.zeros((16,), jnp.float32)

        def zero_chunk(ch):
            def zrow(i, carry):
                for cc in range(D // 16):
                    out_all[i, pl.ds(cc * 16, 16)] = zeros_f
                return carry

            lax.fori_loop(ch * CHUNK, (ch + 1) * CHUNK, zrow, 0)

        def issue(ch, sem):
            dst = out_all.at[pl.ds(ch * CHUNK, CHUNK)]
            for g in range(deg):
                pltpu.async_copy(x_hbm.at[idxT.at[ch, g]], dst, sem, add=True)

        def drain(ch, sem):
            dst = out_all.at[pl.ds(ch * CHUNK, CHUNK)]
            for g in range(deg):
                pltpu.make_async_copy(x_hbm.at[idxT.at[ch, g]], dst,
                                      sem).wait()

        zero_chunk(0)

        @pl.when(is_hi)
        def _():
            pltpu.make_async_copy(nt_hbm.at[pl.ds(chunk0, ch_hi)], idxT,
                                  ssem).wait()

        @pl.when(jnp.logical_not(is_hi))
        def _():
            pltpu.make_async_copy(nt_hbm.at[pl.ds(chunk0, ch_base)],
                                  idxT.at[pl.ds(0, ch_base)], ssem).wait()

        for ch in range(ch_hi):
            @pl.when(ch < chunks_w)
            def _(ch=ch):
                if ch > 0:
                    zero_chunk(ch)
                issue(ch, sems[ch])

        for ch in range(ch_hi):
            @pl.when(ch < chunks_w)
            def _(ch=ch):
                drain(ch, sems[ch])
                pltpu.async_copy(out_all.at[pl.ds(ch * CHUNK, CHUNK)],
                                 s_hbm.at[pl.ds(base + ch * CHUNK, CHUNK)],
                                 fsem)

        for ch in range(ch_hi):
            @pl.when(ch < chunks_w)
            def _(ch=ch):
                pltpu.make_async_copy(
                    out_all.at[pl.ds(ch * CHUNK, CHUNK)],
                    s_hbm.at[pl.ds(base + ch * CHUNK, CHUNK)], fsem).wait()

    return segsum


def _tc_fuse_body(x_ref, s_ref, wg_ref, wc_ref, b_ref, o_ref):
    dn = (((1,), (1,)), ((), ()))
    o = lax.dot_general(x_ref[...], wg_ref[...], dn,
                        preferred_element_type=jnp.float32)
    o = o + lax.dot_general(s_ref[...], wc_ref[...], dn,
                            preferred_element_type=jnp.float32)
    o = o + b_ref[...]
    o_ref[...] = jnp.where(o > 0, o, jnp.exp(jnp.minimum(o, 0.0)) - 1.0)


def _tc_fuse(x, S, Wg, Wc, bias2d):
    N, D = x.shape
    DO = Wg.shape[0]
    blk = 2000
    return pl.pallas_call(
        _tc_fuse_body,
        grid=(N // blk,),
        in_specs=[
            pl.BlockSpec((blk, D), lambda i: (i, 0)),
            pl.BlockSpec((blk, D), lambda i: (i, 0)),
            pl.BlockSpec((DO, D), lambda i: (0, 0)),
            pl.BlockSpec((DO, D), lambda i: (0, 0)),
            pl.BlockSpec((1, DO), lambda i: (0, 0)),
        ],
        out_specs=pl.BlockSpec((blk, DO), lambda i: (i, 0)),
        out_shape=jax.ShapeDtypeStruct((N, DO), jnp.float32),
    )(x, S, Wg, Wc, bias2d)


def kernel(x, W_global, W_local, bias, node_ids, neigh_ids, deg):
    N, D = x.shape
    E = neigh_ids.shape[0]
    deg_static = E // N
    nT = (neigh_ids.astype(jnp.int32)
          .reshape(N // CHUNK, CHUNK, deg_static)
          .transpose(0, 2, 1))
    segsum = _make_segsum_sc(N, D, deg_static)
    S = segsum(x, nT)
    inv_deg = 1.0 / jnp.asarray(deg, jnp.float32)
    Wc = (W_local + W_global) * inv_deg
    out = _tc_fuse(x, S, W_global, Wc, bias.reshape(1, -1))
    return out
